# baseline (speedup 1.0000x reference)
import functools

import jax
import jax.numpy as jnp
from jax import lax
from jax.experimental import pallas as pl
from jax.experimental.pallas import tpu as pltpu
from jax.experimental.pallas import tpu_sc as plsc

_BETA = 0.25


def _argmin_body(z_ref, cb_ref, idx_ref, loss_ref, acc_ref, c2_ref, col_ref):
    i = pl.program_id(0)
    nsteps = pl.num_programs(0)

    z = z_ref[...]
    k = cb_ref.shape[0]

    @pl.when(i == 0)
    def _precompute():
        cb = cb_ref[...]
        c2_ref[...] = jnp.sum(cb * cb, axis=1)[None, :]
        col_ref[...] = lax.broadcasted_iota(jnp.int32, (1, k), 1).astype(
            jnp.float32)

    z2 = jnp.sum(z * z, axis=1, keepdims=True)

    zz = z + z
    zc2 = lax.dot_general(zz, cb_ref[...], (((1,), (1,)), ((), ())))
    dist = z2 + c2_ref[...] - zc2

    m = jnp.min(dist, axis=1, keepdims=True)
    idx_f = jnp.min(jnp.where(dist <= m, col_ref[...], float(k)), axis=1)
    idx_ref[...] = idx_f.astype(jnp.int32)

    @pl.when(i == 0)
    def _init():
        acc_ref[0] = 0.0

    acc_ref[0] += jnp.sum(m)

    @pl.when(i == nsteps - 1)
    def _fini():
        n_total = nsteps * z.shape[0]
        scale = (1.0 + _BETA) / (n_total * z.shape[1])
        loss_ref[...] = (acc_ref[0] * scale).reshape(1, 1)


def _argmin_call(z, codebook, tn):
    n, d = z.shape
    k = codebook.shape[0]
    return pl.pallas_call(
        _argmin_body,
        grid=(n // tn,),
        in_specs=[
            pl.BlockSpec((tn, d), lambda i: (i, 0)),
            pl.BlockSpec((k, d), lambda i: (0, 0)),
        ],
        out_specs=[
            pl.BlockSpec((tn,), lambda i: (i,)),
            pl.BlockSpec((1, 1), lambda i: (0, 0)),
        ],
        out_shape=[
            jax.ShapeDtypeStruct((n,), jnp.int32),
            jax.ShapeDtypeStruct((1, 1), jnp.float32),
        ],
        scratch_shapes=[
            pltpu.SMEM((1,), jnp.float32),
            pltpu.VMEM((1, k), jnp.float32),
            pltpu.VMEM((1, k), jnp.float32),
        ],
    )(z, codebook)


@functools.cache
def _make_sc_gather(v, d, b):
    info = plsc.get_sparse_core_info()
    nc, ns = info.num_cores, info.num_subcores
    nw = nc * ns
    b_per_w = b // nw
    mesh = plsc.VectorSubcoreMesh(core_axis_name="c", subcore_axis_name="s")

    @functools.partial(
        pl.kernel,
        mesh=mesh,
        out_type=jax.ShapeDtypeStruct((b, d), jnp.float32),
        scratch_types=[
            pltpu.VMEM((b_per_w,), jnp.int32),
            pltpu.VMEM((b_per_w, d), jnp.float32),
            pltpu.SemaphoreType.DMA,
        ],
    )
    def gather_kernel(table_hbm, idx_hbm, out_hbm, idx_v, rows_v, sem):
        wid = lax.axis_index("s") * nc + lax.axis_index("c")
        base = wid * b_per_w
        pltpu.sync_copy(idx_hbm.at[pl.ds(base, b_per_w)], idx_v)
        pltpu.async_copy(table_hbm.at[idx_v], rows_v, sem).wait()
        pltpu.sync_copy(rows_v, out_hbm.at[pl.ds(base, b_per_w)])

    return gather_kernel


def kernel(x, codebook):
    b, d, h, w = x.shape
    k = codebook.shape[0]
    z_flat = jnp.transpose(x, (0, 2, 3, 1)).reshape(-1, d)
    n = z_flat.shape[0]

    indices, loss = _argmin_call(z_flat, codebook, tn=1024)
    z_q_flat = _make_sc_gather(k, d, n)(codebook, indices)
    z_q = jnp.transpose(z_q_flat.reshape(b, h, w, d), (0, 3, 1, 2))
    return (z_q, loss.reshape(()), indices)


# speedup vs baseline: 1.1288x; 1.1288x over previous
import functools

import jax
import jax.numpy as jnp
from jax import lax
from jax.experimental import pallas as pl
from jax.experimental.pallas import tpu as pltpu
from jax.experimental.pallas import tpu_sc as plsc

_BETA = 0.25


def _argmin_body(z_ref, cb_ref, idx_ref, loss_ref, acc_ref, c2_ref, col_ref):
    i = pl.program_id(0)
    nsteps = pl.num_programs(0)

    z = z_ref[...]
    k = cb_ref.shape[0]

    @pl.when(i == 0)
    def _precompute():
        cb = cb_ref[...]
        c2_ref[...] = jnp.sum(cb * cb, axis=1)[None, :]
        col_ref[...] = lax.broadcasted_iota(jnp.int32, (1, k), 1).astype(
            jnp.float32)

    z2 = jnp.sum(z * z, axis=1, keepdims=True)

    zc = lax.dot_general(z, cb_ref[...], (((1,), (1,)), ((), ())))
    dist = z2 + c2_ref[...] - 2.0 * zc

    m = jnp.min(dist, axis=1, keepdims=True)
    idx_f = jnp.min(jnp.where(dist <= m, col_ref[...], float(k)), axis=1)
    idx_ref[...] = idx_f.astype(jnp.int32)

    @pl.when(i == 0)
    def _init():
        acc_ref[0] = 0.0

    acc_ref[0] += jnp.sum(m)

    @pl.when(i == nsteps - 1)
    def _fini():
        n_total = nsteps * z.shape[0]
        scale = (1.0 + _BETA) / (n_total * z.shape[1])
        loss_ref[...] = (acc_ref[0] * scale).reshape(1, 1)


def _argmin_call(z, codebook, tn):
    n, d = z.shape
    k = codebook.shape[0]
    return pl.pallas_call(
        _argmin_body,
        grid=(n // tn,),
        in_specs=[
            pl.BlockSpec((tn, d), lambda i: (i, 0)),
            pl.BlockSpec((k, d), lambda i: (0, 0)),
        ],
        out_specs=[
            pl.BlockSpec((tn,), lambda i: (i,)),
            pl.BlockSpec((1, 1), lambda i: (0, 0)),
        ],
        out_shape=[
            jax.ShapeDtypeStruct((n,), jnp.int32),
            jax.ShapeDtypeStruct((1, 1), jnp.float32),
        ],
        scratch_shapes=[
            pltpu.SMEM((1,), jnp.float32),
            pltpu.VMEM((1, k), jnp.float32),
            pltpu.VMEM((1, k), jnp.float32),
        ],
    )(z, codebook)


@functools.cache
def _make_sc_gather(v, d, b):
    info = plsc.get_sparse_core_info()
    nc, ns = info.num_cores, info.num_subcores
    nw = nc * ns
    b_per_w = b // nw
    mesh = plsc.VectorSubcoreMesh(core_axis_name="c", subcore_axis_name="s")

    hb = b_per_w // 2

    @functools.partial(
        pl.kernel,
        mesh=mesh,
        out_type=jax.ShapeDtypeStruct((b, d), jnp.float32),
        scratch_types=[
            pltpu.VMEM((b_per_w,), jnp.int32),
            pltpu.VMEM((b_per_w, d), jnp.float32),
            pltpu.SemaphoreType.DMA,
            pltpu.SemaphoreType.DMA,
            pltpu.SemaphoreType.DMA,
            pltpu.SemaphoreType.DMA,
        ],
    )
    def gather_kernel(table_hbm, idx_hbm, out_hbm, idx_v, rows_v,
                      g0, g1, w0, w1):
        wid = lax.axis_index("s") * nc + lax.axis_index("c")
        base = wid * b_per_w
        pltpu.sync_copy(idx_hbm.at[pl.ds(base, b_per_w)], idx_v)
        c0 = pltpu.async_copy(
            table_hbm.at[idx_v.at[pl.ds(0, hb)]], rows_v.at[pl.ds(0, hb)], g0)
        c1 = pltpu.async_copy(
            table_hbm.at[idx_v.at[pl.ds(hb, hb)]], rows_v.at[pl.ds(hb, hb)],
            g1)
        c0.wait()
        s0 = pltpu.async_copy(
            rows_v.at[pl.ds(0, hb)], out_hbm.at[pl.ds(base, hb)], w0)
        c1.wait()
        s1 = pltpu.async_copy(
            rows_v.at[pl.ds(hb, hb)], out_hbm.at[pl.ds(base + hb, hb)], w1)
        s0.wait()
        s1.wait()

    return gather_kernel


def kernel(x, codebook):
    b, d, h, w = x.shape
    k = codebook.shape[0]
    z_flat = jnp.transpose(x, (0, 2, 3, 1)).reshape(-1, d)
    n = z_flat.shape[0]

    indices, loss = _argmin_call(z_flat, codebook, tn=1024)
    z_q_flat = _make_sc_gather(k, d, n)(codebook, indices)
    z_q = jnp.transpose(z_q_flat.reshape(b, h, w, d), (0, 3, 1, 2))
    return (z_q, loss.reshape(()), indices)
